# baseline (speedup 1.0000x reference)
import math

import jax
import jax.numpy as jnp
from jax.experimental import pallas as pl
from jax.experimental.pallas import tpu as pltpu

_DIM = 2048
_H = 16
_DH = 128
_S = 2048
_KEEP = max(1, int(_S * (1.0 - 0.6)))
_BLK = 256
_NB = _S // _BLK
_SCALE = 1.0 / math.sqrt(_DH)
_NEG = -1e9


def _key_to_f32(cand_u):
    return jax.lax.bitcast_convert_type(
        jnp.where(cand_u >= jnp.uint32(0x80000000),
                  cand_u & jnp.uint32(0x7FFFFFFF), ~cand_u),
        jnp.float32)


def _attn_kernel(x_ref, wq_ref, bq_ref, wk_ref, bk_ref, wv_ref, bv_ref,
                 o_ref, k_s, v_s, s_s, o_acc):
    i = pl.program_id(1)
    npan = i + 1

    @pl.when(i == 0)
    def _():
        xh = x_ref[...]
        k_s[...] = (jnp.dot(xh, wk_ref[0], preferred_element_type=jnp.float32)
                    + bk_ref[0]).reshape(_NB, _BLK, _DH)
        v_s[...] = (jnp.dot(xh, wv_ref[0], preferred_element_type=jnp.float32)
                    + bv_ref[0]).reshape(_NB, _BLK, _DH)

    xq = x_ref[pl.ds(i * _BLK, _BLK), :]
    q = jnp.dot(xq, wq_ref[0], preferred_element_type=jnp.float32) + bq_ref[0]

    def sc_body(j, m):
        s = jax.lax.dot_general(q, k_s[j], (((1,), (1,)), ((), ())),
                                preferred_element_type=jnp.float32) * _SCALE
        s_s[j] = s
        return jnp.maximum(m, jnp.max(s, axis=1, keepdims=True))

    m = jax.lax.fori_loop(0, i, sc_body,
                          jnp.full((_BLK, 1), _NEG, jnp.float32))
    sd = jax.lax.dot_general(q, k_s[i], (((1,), (1,)), ((), ())),
                             preferred_element_type=jnp.float32) * _SCALE
    lrow = jax.lax.broadcasted_iota(jnp.int32, (_BLK, _BLK), 0)
    lcol = jax.lax.broadcasted_iota(jnp.int32, (_BLK, _BLK), 1)
    sd = jnp.where(lcol <= lrow, sd, _NEG)
    s_s[i] = sd
    m = jnp.maximum(m, jnp.max(sd, axis=1, keepdims=True))

    prefix = jnp.zeros((_BLK, 1), jnp.uint32)
    for bit in range(31, -1, -1):
        fc = _key_to_f32(prefix | jnp.uint32(1 << bit))

        def cnt_body(j, acc, fc=fc):
            return acc + jnp.sum((s_s[j] >= fc).astype(jnp.float32),
                                 axis=1, keepdims=True)

        cnt = jax.lax.fori_loop(0, npan, cnt_body,
                                jnp.zeros((_BLK, 1), jnp.float32))
        prefix = jnp.where(cnt >= float(_KEEP),
                           prefix | jnp.uint32(1 << bit), prefix)
    thr = jnp.where(prefix == 0, jnp.float32(_NEG), _key_to_f32(prefix))

    o_acc[...] = jnp.zeros((_BLK, _DH), jnp.float32)

    def av_body(j, denom):
        s = s_s[j]
        p = jnp.where(s >= thr, jnp.exp(s - m), 0.0)
        o_acc[...] += jnp.dot(p, v_s[j], preferred_element_type=jnp.float32)
        return denom + jnp.sum(p, axis=1, keepdims=True)

    denom = jax.lax.fori_loop(0, npan, av_body,
                              jnp.zeros((_BLK, 1), jnp.float32))
    o_ref[0] = o_acc[...] / denom


def _attention(x2, wq, bq3, wk, bk3, wv, bv3):
    w_spec = pl.BlockSpec((1, _DH, _DH), lambda h, i: (h, 0, 0))
    b_spec = pl.BlockSpec((1, 1, _DH), lambda h, i: (h, 0, 0))
    return pl.pallas_call(
        _attn_kernel,
        grid=(_H, _NB),
        in_specs=[
            pl.BlockSpec((_S, _DH), lambda h, i: (0, h)),
            w_spec, b_spec, w_spec, b_spec, w_spec, b_spec,
        ],
        out_specs=pl.BlockSpec((1, _BLK, _DH), lambda h, i: (h, i, 0)),
        out_shape=jax.ShapeDtypeStruct((_H, _S, _DH), jnp.float32),
        scratch_shapes=[
            pltpu.VMEM((_NB, _BLK, _DH), jnp.float32),
            pltpu.VMEM((_NB, _BLK, _DH), jnp.float32),
            pltpu.VMEM((_NB, _BLK, _BLK), jnp.float32),
            pltpu.VMEM((_BLK, _DH), jnp.float32),
        ],
    )(x2, wq, bq3, wk, bk3, wv, bv3)


_TM = 256
_TN = 256


def _proj_kernel(a_ref, wo_ref, bo_ref, o_ref):
    o_ref[...] = jax.lax.dot_general(
        a_ref[...], wo_ref[...], (((1,), (1,)), ((), ())),
        preferred_element_type=jnp.float32) + bo_ref[0]


def _proj(a, wo, bo2):
    return pl.pallas_call(
        _proj_kernel,
        grid=(_DIM // _TN, _S // _TM),
        in_specs=[
            pl.BlockSpec((_TM, _DIM), lambda tj, ti: (ti, 0)),
            pl.BlockSpec((_TN, _DIM), lambda tj, ti: (tj, 0)),
            pl.BlockSpec((1, _TN), lambda tj, ti: (0, tj)),
        ],
        out_specs=pl.BlockSpec((_TM, _TN), lambda tj, ti: (ti, tj)),
        out_shape=jax.ShapeDtypeStruct((_S, _DIM), jnp.float32),
    )(a, wo, bo2)


def kernel(x, causal_mask, Wq, bq, Wk, bk, Wv, bv, Wo, bo):
    x2 = x.reshape(_S, _DIM)
    out = _attention(
        x2, Wq, bq.reshape(_H, 1, _DH),
        Wk, bk.reshape(_H, 1, _DH),
        Wv, bv.reshape(_H, 1, _DH))
    a = out.transpose(0, 2, 1).reshape(_DIM, _S)
    final = _proj(a, Wo, bo.reshape(1, _DIM))
    return final.reshape(1, _S, _DIM)


# speedup vs baseline: 3.5621x; 3.5621x over previous
import math

import jax
import jax.numpy as jnp
from jax.experimental import pallas as pl

_DIM = 2048
_H = 16
_DH = 128
_S = 2048
_KEEP = max(1, int(_S * (1.0 - 0.6)))
_BLK = 256
_NB = _S // _BLK
_SCALE = 1.0 / math.sqrt(_DH)
_NEG = -1e9


def _key_to_f32(cand_u):
    return jax.lax.bitcast_convert_type(
        jnp.where(cand_u >= jnp.uint32(0x80000000),
                  cand_u & jnp.uint32(0x7FFFFFFF), ~cand_u),
        jnp.float32)


def _qkv_kernel(x_ref, wq_ref, bq_ref, wk_ref, bk_ref, wv_ref, bv_ref,
                q_ref, k_ref, v_ref):
    xh = x_ref[...]
    q_ref[0] = jnp.dot(xh, wq_ref[0], preferred_element_type=jnp.float32) + bq_ref[0]
    k_ref[0] = jnp.dot(xh, wk_ref[0], preferred_element_type=jnp.float32) + bk_ref[0]
    v_ref[0] = jnp.dot(xh, wv_ref[0], preferred_element_type=jnp.float32) + bv_ref[0]


def _qkv(x2, wq, bq3, wk, bk3, wv, bv3):
    w_spec = pl.BlockSpec((1, _DH, _DH), lambda h: (h, 0, 0))
    b_spec = pl.BlockSpec((1, 1, _DH), lambda h: (h, 0, 0))
    qkv_shape = jax.ShapeDtypeStruct((_H, _S, _DH), jnp.float32)
    qkv_spec = pl.BlockSpec((1, _S, _DH), lambda h: (h, 0, 0))
    return pl.pallas_call(
        _qkv_kernel,
        grid=(_H,),
        in_specs=[
            pl.BlockSpec((_S, _DH), lambda h: (0, h)),
            w_spec, b_spec, w_spec, b_spec, w_spec, b_spec,
        ],
        out_specs=(qkv_spec, qkv_spec, qkv_spec),
        out_shape=(qkv_shape, qkv_shape, qkv_shape),
    )(x2, wq, bq3, wk, bk3, wv, bv3)


def _mk_attn_block(i):
    ncol = (i + 1) * _BLK
    do_search = ncol >= _KEEP

    def body(q_ref, k_ref, v_ref, o_ref):
        q = q_ref[0]
        scores = jax.lax.dot_general(
            q, k_ref[0], (((1,), (1,)), ((), ())),
            preferred_element_type=jnp.float32) * _SCALE
        rows = i * _BLK + jax.lax.broadcasted_iota(jnp.int32, (_BLK, ncol), 0)
        cols = jax.lax.broadcasted_iota(jnp.int32, (_BLK, ncol), 1)
        scores = jnp.where(cols <= rows, scores, _NEG)
        m = jnp.max(scores, axis=1, keepdims=True)
        p = jnp.exp(scores - m)
        if do_search:
            prefix = jnp.zeros((_BLK, 1), jnp.uint32)
            for bit in range(31, -1, -1):
                fc = _key_to_f32(prefix | jnp.uint32(1 << bit))
                cnt = jnp.sum((scores >= fc).astype(jnp.float32),
                              axis=1, keepdims=True)
                prefix = jnp.where(cnt >= float(_KEEP),
                                   prefix | jnp.uint32(1 << bit), prefix)
            p = jnp.where(scores >= _key_to_f32(prefix), p, 0.0)
        denom = jnp.sum(p, axis=1, keepdims=True)
        o_ref[0] = jnp.dot(p, v_ref[0],
                           preferred_element_type=jnp.float32) / denom

    def call(q_all, k_all, v_all):
        return pl.pallas_call(
            body,
            grid=(_H,),
            in_specs=[
                pl.BlockSpec((1, _BLK, _DH), lambda h: (h, i, 0)),
                pl.BlockSpec((1, ncol, _DH), lambda h: (h, 0, 0)),
                pl.BlockSpec((1, ncol, _DH), lambda h: (h, 0, 0)),
            ],
            out_specs=pl.BlockSpec((1, _BLK, _DH), lambda h: (h, 0, 0)),
            out_shape=jax.ShapeDtypeStruct((_H, _BLK, _DH), jnp.float32),
        )(q_all, k_all, v_all)

    return call


_ATTN_BLOCKS = [_mk_attn_block(i) for i in range(_NB)]

_TM = 256
_TN = 256


def _proj_kernel(a_ref, wo_ref, bo_ref, o_ref):
    o_ref[...] = jax.lax.dot_general(
        a_ref[...], wo_ref[...], (((1,), (1,)), ((), ())),
        preferred_element_type=jnp.float32) + bo_ref[0]


def _proj(a, wo, bo2):
    return pl.pallas_call(
        _proj_kernel,
        grid=(_DIM // _TN, _S // _TM),
        in_specs=[
            pl.BlockSpec((_TM, _DIM), lambda tj, ti: (ti, 0)),
            pl.BlockSpec((_TN, _DIM), lambda tj, ti: (tj, 0)),
            pl.BlockSpec((1, _TN), lambda tj, ti: (0, tj)),
        ],
        out_specs=pl.BlockSpec((_TM, _TN), lambda tj, ti: (ti, tj)),
        out_shape=jax.ShapeDtypeStruct((_S, _DIM), jnp.float32),
    )(a, wo, bo2)


def kernel(x, causal_mask, Wq, bq, Wk, bk, Wv, bv, Wo, bo):
    x2 = x.reshape(_S, _DIM)
    q_all, k_all, v_all = _qkv(
        x2, Wq, bq.reshape(_H, 1, _DH),
        Wk, bk.reshape(_H, 1, _DH),
        Wv, bv.reshape(_H, 1, _DH))
    out = jnp.concatenate(
        [blk(q_all, k_all, v_all) for blk in _ATTN_BLOCKS], axis=1)
    a = out.transpose(0, 2, 1).reshape(_DIM, _S)
    final = _proj(a, Wo, bo.reshape(1, _DIM))
    return final.reshape(1, _S, _DIM)


# speedup vs baseline: 3.6204x; 1.0164x over previous
import math

import jax
import jax.numpy as jnp
from jax.experimental import pallas as pl
from jax.experimental.pallas import tpu as pltpu

_DIM = 2048
_H = 16
_DH = 128
_S = 2048
_KEEP = max(1, int(_S * (1.0 - 0.6)))
_BLK = 256
_NB = _S // _BLK
_SCALE = 1.0 / math.sqrt(_DH)
_NEG = -1e9


def _key_to_f32(cand_u):
    return jax.lax.bitcast_convert_type(
        jnp.where(cand_u >= jnp.uint32(0x80000000),
                  cand_u & jnp.uint32(0x7FFFFFFF), ~cand_u),
        jnp.float32)


def _mono_kernel(x_ref, wq_ref, bq_ref, wk_ref, bk_ref, wv_ref, bv_ref,
                 wo_ref, bo_ref, o_ref, k_s, v_s, o_s):
    xh = x_ref[...]
    k_s[...] = jnp.dot(xh, wk_ref[0], preferred_element_type=jnp.float32) + bk_ref[0]
    v_s[...] = jnp.dot(xh, wv_ref[0], preferred_element_type=jnp.float32) + bv_ref[0]

    for i in range(_NB):
        ncol = (i + 1) * _BLK
        q = jnp.dot(xh[i * _BLK:(i + 1) * _BLK, :], wq_ref[0],
                    preferred_element_type=jnp.float32) + bq_ref[0]
        scores = jax.lax.dot_general(
            q, k_s[0:ncol, :], (((1,), (1,)), ((), ())),
            preferred_element_type=jnp.float32) * _SCALE
        rows = i * _BLK + jax.lax.broadcasted_iota(jnp.int32, (_BLK, ncol), 0)
        cols = jax.lax.broadcasted_iota(jnp.int32, (_BLK, ncol), 1)
        scores = jnp.where(cols <= rows, scores, _NEG)
        m = jnp.max(scores, axis=1, keepdims=True)
        p = jnp.exp(scores - m)
        if ncol >= _KEEP:
            prefix = jnp.zeros((_BLK, 1), jnp.uint32)
            for bit in range(31, -1, -1):
                fc = _key_to_f32(prefix | jnp.uint32(1 << bit))
                cnt = jnp.sum((scores >= fc).astype(jnp.float32),
                              axis=1, keepdims=True)
                prefix = jnp.where(cnt >= float(_KEEP),
                                   prefix | jnp.uint32(1 << bit), prefix)
            p = jnp.where(scores >= _key_to_f32(prefix), p, 0.0)
        denom = jnp.sum(p, axis=1, keepdims=True)
        o_s[i * _BLK:(i + 1) * _BLK, :] = jnp.dot(
            p, v_s[0:ncol, :], preferred_element_type=jnp.float32) / denom

    o_ref[...] = jax.lax.dot_general(
        o_s[...], wo_ref[...], (((0,), (1,)), ((), ())),
        preferred_element_type=jnp.float32) + bo_ref[0]


def kernel(x, causal_mask, Wq, bq, Wk, bk, Wv, bv, Wo, bo):
    x2 = x.reshape(_S, _DIM)
    w_spec = pl.BlockSpec((1, _DH, _DH), lambda h: (h, 0, 0))
    b_spec = pl.BlockSpec((1, 1, _DH), lambda h: (h, 0, 0))
    final = pl.pallas_call(
        _mono_kernel,
        grid=(_H,),
        in_specs=[
            pl.BlockSpec((_S, _DH), lambda h: (0, h)),
            w_spec, b_spec, w_spec, b_spec, w_spec, b_spec,
            pl.BlockSpec((_DIM, _DIM), lambda h: (0, 0)),
            pl.BlockSpec((1, _DIM), lambda h: (0, 0)),
        ],
        out_specs=pl.BlockSpec((_DH, _DIM), lambda h: (h, 0)),
        out_shape=jax.ShapeDtypeStruct((_S, _DIM), jnp.float32),
        scratch_shapes=[
            pltpu.VMEM((_S, _DH), jnp.float32),
            pltpu.VMEM((_S, _DH), jnp.float32),
            pltpu.VMEM((_S, _DH), jnp.float32),
        ],
    )(x2, Wq, bq.reshape(_H, 1, _DH), Wk, bk.reshape(_H, 1, _DH),
      Wv, bv.reshape(_H, 1, _DH), Wo, bo.reshape(1, _DIM))
    return final.reshape(1, _S, _DIM)
